# baseline (speedup 1.0000x reference)
import functools

import jax
import jax.numpy as jnp
from jax import lax
from jax.experimental import pallas as pl
from jax.experimental.pallas import tpu as pltpu
from jax.experimental.pallas import tpu_sc as plsc

_VOCAB = 1000000
_EMBED = 64
_MAXLEN = 200
_LABELS = 64
_BATCH = 4096

_NC, _NS = 2, 16
_NW = _NC * _NS
_BPW = _BATCH // _NW
_C0, _C1 = 104, 96

_W = 7680
_NBLK = -(-_VOCAB // _W)
_NPAIR = -(-_NBLK // 2)
_PROWS = _NPAIR * _W


def _proj_body(tlo_ref, thi_ref, w_ref, o_ref):
    t2 = jnp.concatenate([tlo_ref[...], thi_ref[...]], axis=0)
    o_ref[...] = lax.dot_general(
        t2, w_ref[...], (((0,), (0,)), ((), ())),
        preferred_element_type=jnp.float32)


def _project(table_t, w_blk):
    return pl.pallas_call(
        _proj_body,
        grid=(_NPAIR,),
        in_specs=[
            pl.BlockSpec((_EMBED, _W), lambda i: (0, 2 * i)),
            pl.BlockSpec(
                (_EMBED, _W),
                lambda i: (0, jnp.minimum(2 * i + 1, _NBLK - 1))),
            pl.BlockSpec((2 * _EMBED, 2 * _EMBED), lambda i: (0, 0)),
        ],
        out_specs=pl.BlockSpec((_W, 2 * _EMBED), lambda i: (i, 0)),
        out_shape=jax.ShapeDtypeStruct((_PROWS, 2 * _EMBED), jnp.float32),
    )(table_t, table_t, w_blk)


def _make_mean_kernel():
    mesh = plsc.VectorSubcoreMesh(core_axis_name="c", subcore_axis_name="s")

    @functools.partial(
        pl.kernel,
        out_type=jax.ShapeDtypeStruct((_BATCH, _LABELS), jnp.float32),
        mesh=mesh,
        compiler_params=pltpu.CompilerParams(use_tc_tiling_on_sc=False),
        scratch_types=[
            pltpu.VMEM((_MAXLEN, _BPW), jnp.int32),
            pltpu.VMEM((4, _BPW, _LABELS), jnp.float32),
            pltpu.VMEM((_BPW, _LABELS), jnp.float32),
            pltpu.VMEM((_LABELS,), jnp.float32),
            pltpu.SemaphoreType.DMA,
            pltpu.SemaphoreType.DMA,
            pltpu.SemaphoreType.DMA,
            pltpu.SemaphoreType.DMA,
        ],
    )
    def mean_kernel(x_hbm, p_hbm, b_hbm, out_hbm,
                    idx_v, acc_v, out_v, b_v, sem0, sem1, sem2, sem3):
        wid = lax.axis_index("s") * _NC + lax.axis_index("c")
        base = wid * _BPW
        pltpu.sync_copy(b_hbm, b_v)
        pltpu.sync_copy(x_hbm.at[:, pl.ds(base, _BPW)], idx_v)

        inv_len = jnp.float32(1.0 / _MAXLEN)
        zero = jnp.zeros((16,), jnp.float32)

        def zero_body(i, carry):
            for q in range(4):
                for c in range(4):
                    acc_v[q, i, pl.ds(c * 16, 16)] = zero
            return carry

        lax.fori_loop(0, _BPW, zero_body, 0)

        def fire(l, buf, sem):
            pltpu.async_copy(
                p_hbm.at[idx_v.at[l]], acc_v.at[buf], sem, add=True)

        def wait(buf, sem):
            pltpu.make_async_copy(
                p_hbm.at[idx_v.at[0]], acc_v.at[buf], sem).wait()

        sems = (sem0, sem1, sem2, sem3)
        for u in range(4):
            fire(u, u, sems[u])

        def quad_body(ll, carry):
            l0 = 4 + ll * 4
            for u in range(4):
                wait(u, sems[u])
                fire(l0 + u, u, sems[u])
            return carry

        lax.fori_loop(0, (_MAXLEN - 4) // 4, quad_body, 0)
        for u in range(4):
            wait(u, sems[u])

        bias = tuple(b_v[pl.ds(c * 16, 16)] for c in range(4))

        def finish_body(i, carry):
            for c in range(4):
                s = ((acc_v[0, i, pl.ds(c * 16, 16)]
                      + acc_v[1, i, pl.ds(c * 16, 16)])
                     + (acc_v[2, i, pl.ds(c * 16, 16)]
                        + acc_v[3, i, pl.ds(c * 16, 16)]))
                out_v[i, pl.ds(c * 16, 16)] = s * inv_len + bias[c]
            return carry

        lax.fori_loop(0, _BPW, finish_body, 0)
        pltpu.sync_copy(out_v, out_hbm.at[pl.ds(base, _BPW)])

    return mean_kernel


_MEAN_KERNEL = _make_mean_kernel()


def kernel(x, table, fc_w, fc_b):
    wt = fc_w.T
    z = jnp.zeros((_EMBED, _EMBED), jnp.float32)
    w_blk = jnp.block([[wt, z], [z, wt]])
    p_packed = _project(table.T, w_blk)
    p_rows = p_packed.reshape(2 * _PROWS, _LABELS)
    b = x // _W
    s = x - b * _W
    x2 = (((b >> 1) * _W + s) << 1) + (b & 1)
    return _MEAN_KERNEL(x2.T, p_rows, fc_b)


# speedup vs baseline: 1.0441x; 1.0441x over previous
import functools

import jax
import jax.numpy as jnp
from jax import lax
from jax.experimental import pallas as pl
from jax.experimental.pallas import tpu as pltpu
from jax.experimental.pallas import tpu_sc as plsc

_VOCAB = 1000000
_EMBED = 64
_MAXLEN = 200
_LABELS = 64
_BATCH = 4096

_NC, _NS = 2, 16
_NW = _NC * _NS
_BPW = _BATCH // _NW
_C0, _C1 = 104, 96

_W = 7680
_NBLK = -(-_VOCAB // _W)
_NPAIR = -(-_NBLK // 2)
_PROWS = _NPAIR * _W


def _proj_body(tlo_ref, thi_ref, w_ref, o_ref):
    t2 = jnp.concatenate([tlo_ref[...], thi_ref[...]], axis=0)
    o_ref[...] = lax.dot_general(
        t2, w_ref[...], (((0,), (0,)), ((), ())),
        preferred_element_type=jnp.float32)


def _project(table_t, w_blk):
    return pl.pallas_call(
        _proj_body,
        grid=(_NPAIR,),
        in_specs=[
            pl.BlockSpec((_EMBED, _W), lambda i: (0, 2 * i)),
            pl.BlockSpec(
                (_EMBED, _W),
                lambda i: (0, jnp.minimum(2 * i + 1, _NBLK - 1))),
            pl.BlockSpec((2 * _EMBED, 2 * _EMBED), lambda i: (0, 0)),
        ],
        out_specs=pl.BlockSpec((_W, 2 * _EMBED), lambda i: (i, 0)),
        out_shape=jax.ShapeDtypeStruct((_PROWS, 2 * _EMBED), jnp.float32),
    )(table_t, table_t, w_blk)


def _make_mean_kernel():
    mesh = plsc.VectorSubcoreMesh(core_axis_name="c", subcore_axis_name="s")

    @functools.partial(
        pl.kernel,
        out_type=jax.ShapeDtypeStruct((_BATCH, _LABELS), jnp.float32),
        mesh=mesh,
        compiler_params=pltpu.CompilerParams(use_tc_tiling_on_sc=False),
        scratch_types=[
            pltpu.VMEM((_MAXLEN, _BPW), jnp.int32),
            pltpu.VMEM((8, _BPW, _LABELS), jnp.float32),
            pltpu.VMEM((_BPW, _LABELS), jnp.float32),
            pltpu.VMEM((_LABELS,), jnp.float32),
            pltpu.SemaphoreType.DMA,
            pltpu.SemaphoreType.DMA,
            pltpu.SemaphoreType.DMA,
            pltpu.SemaphoreType.DMA,
            pltpu.SemaphoreType.DMA,
            pltpu.SemaphoreType.DMA,
            pltpu.SemaphoreType.DMA,
            pltpu.SemaphoreType.DMA,
        ],
    )
    def mean_kernel(x_hbm, p_hbm, b_hbm, out_hbm,
                    idx_v, acc_v, out_v, b_v,
                    sem0, sem1, sem2, sem3, sem4, sem5, sem6, sem7):
        wid = lax.axis_index("s") * _NC + lax.axis_index("c")
        base = wid * _BPW
        pltpu.sync_copy(b_hbm, b_v)
        pltpu.sync_copy(x_hbm.at[:, pl.ds(base, _BPW)], idx_v)

        inv_len = jnp.float32(1.0 / _MAXLEN)
        zero = jnp.zeros((16,), jnp.float32)

        def zero_body(i, carry):
            for q in range(8):
                for c in range(4):
                    acc_v[q, i, pl.ds(c * 16, 16)] = zero
            return carry

        lax.fori_loop(0, _BPW, zero_body, 0)

        def fire(l, buf, sem):
            pltpu.async_copy(
                p_hbm.at[idx_v.at[l]], acc_v.at[buf], sem, add=True)

        def wait(buf, sem):
            pltpu.make_async_copy(
                p_hbm.at[idx_v.at[0]], acc_v.at[buf], sem).wait()

        sems = (sem0, sem1, sem2, sem3, sem4, sem5, sem6, sem7)
        for u in range(8):
            fire(u, u, sems[u])

        def oct_body(ll, carry):
            l0 = 8 + ll * 8
            for u in range(8):
                wait(u, sems[u])
                fire(l0 + u, u, sems[u])
            return carry

        lax.fori_loop(0, (_MAXLEN - 8) // 8, oct_body, 0)
        for u in range(8):
            wait(u, sems[u])

        bias = tuple(b_v[pl.ds(c * 16, 16)] for c in range(4))

        def finish_body(i, carry):
            for c in range(4):
                s01 = (acc_v[0, i, pl.ds(c * 16, 16)]
                       + acc_v[1, i, pl.ds(c * 16, 16)])
                s23 = (acc_v[2, i, pl.ds(c * 16, 16)]
                       + acc_v[3, i, pl.ds(c * 16, 16)])
                s45 = (acc_v[4, i, pl.ds(c * 16, 16)]
                       + acc_v[5, i, pl.ds(c * 16, 16)])
                s67 = (acc_v[6, i, pl.ds(c * 16, 16)]
                       + acc_v[7, i, pl.ds(c * 16, 16)])
                s = (s01 + s23) + (s45 + s67)
                out_v[i, pl.ds(c * 16, 16)] = s * inv_len + bias[c]
            return carry

        lax.fori_loop(0, _BPW, finish_body, 0)
        pltpu.sync_copy(out_v, out_hbm.at[pl.ds(base, _BPW)])

    return mean_kernel


_MEAN_KERNEL = _make_mean_kernel()


def kernel(x, table, fc_w, fc_b):
    wt = fc_w.T
    z = jnp.zeros((_EMBED, _EMBED), jnp.float32)
    w_blk = jnp.block([[wt, z], [z, wt]])
    p_packed = _project(table.T, w_blk)
    p_rows = p_packed.reshape(2 * _PROWS, _LABELS)
    b = x // _W
    s = x - b * _W
    x2 = (((b >> 1) * _W + s) << 1) + (b & 1)
    return _MEAN_KERNEL(x2.T, p_rows, fc_b)


# speedup vs baseline: 1.0514x; 1.0070x over previous
import functools

import jax
import jax.numpy as jnp
from jax import lax
from jax.experimental import pallas as pl
from jax.experimental.pallas import tpu as pltpu
from jax.experimental.pallas import tpu_sc as plsc

_VOCAB = 1000000
_EMBED = 64
_MAXLEN = 200
_LABELS = 64
_BATCH = 4096

_NC, _NS = 2, 16
_NW = _NC * _NS
_BPW = _BATCH // _NW
_C0, _C1 = 104, 96

_W = 7680
_NBLK = -(-_VOCAB // _W)
_NPAIR = -(-_NBLK // 2)
_PROWS = _NPAIR * _W


def _proj_body(tlo_ref, thi_ref, w_ref, o_ref):
    t2 = jnp.concatenate([tlo_ref[...], thi_ref[...]], axis=0)
    o_ref[...] = lax.dot_general(
        t2, w_ref[...], (((0,), (0,)), ((), ())),
        preferred_element_type=jnp.float32)


def _project(table_t, w_blk):
    return pl.pallas_call(
        _proj_body,
        grid=(_NPAIR,),
        in_specs=[
            pl.BlockSpec((_EMBED, _W), lambda i: (0, 2 * i)),
            pl.BlockSpec(
                (_EMBED, _W),
                lambda i: (0, jnp.minimum(2 * i + 1, _NBLK - 1))),
            pl.BlockSpec((2 * _EMBED, 2 * _EMBED), lambda i: (0, 0)),
        ],
        out_specs=pl.BlockSpec((_W, 2 * _EMBED), lambda i: (i, 0)),
        out_shape=jax.ShapeDtypeStruct((_PROWS, 2 * _EMBED), jnp.float32),
    )(table_t, table_t, w_blk)


def _make_mean_kernel():
    mesh = plsc.VectorSubcoreMesh(core_axis_name="c", subcore_axis_name="s")

    @functools.partial(
        pl.kernel,
        out_type=jax.ShapeDtypeStruct((_BATCH, _LABELS), jnp.float32),
        mesh=mesh,
        compiler_params=pltpu.CompilerParams(use_tc_tiling_on_sc=False),
        scratch_types=[
            pltpu.VMEM((_MAXLEN, _BPW), jnp.int32),
            pltpu.VMEM((8, _BPW, _LABELS), jnp.float32),
            pltpu.VMEM((_BPW, _LABELS), jnp.float32),
            pltpu.VMEM((_LABELS,), jnp.float32),
            pltpu.SemaphoreType.DMA,
            pltpu.SemaphoreType.DMA,
            pltpu.SemaphoreType.DMA,
            pltpu.SemaphoreType.DMA,
            pltpu.SemaphoreType.DMA,
            pltpu.SemaphoreType.DMA,
            pltpu.SemaphoreType.DMA,
            pltpu.SemaphoreType.DMA,
        ],
    )
    def mean_kernel(x_hbm, p_hbm, b_hbm, out_hbm,
                    idx_v, acc_v, out_v, b_v,
                    sem0, sem1, sem2, sem3, sem4, sem5, sem6, sem7):
        wid = lax.axis_index("s") * _NC + lax.axis_index("c")
        base = wid * _BPW
        pltpu.sync_copy(b_hbm, b_v)
        pltpu.sync_copy(x_hbm.at[:, pl.ds(base, _BPW)], idx_v)

        inv_len = jnp.float32(1.0 / _MAXLEN)
        zero = jnp.zeros((16,), jnp.float32)

        def zero_body(i, carry):
            for q in range(8):
                for c in range(4):
                    acc_v[q, i, pl.ds(c * 16, 16)] = zero
            return carry

        lax.fori_loop(0, _BPW, zero_body, 0)

        def fire(l, buf, sem):
            pltpu.async_copy(
                p_hbm.at[idx_v.at[l]], acc_v.at[buf], sem, add=True)

        def wait(buf, sem):
            pltpu.make_async_copy(
                p_hbm.at[idx_v.at[0]], acc_v.at[buf], sem).wait()

        sems = (sem0, sem1, sem2, sem3, sem4, sem5, sem6, sem7)

        def fire_body(ll, carry):
            l0 = ll * 8
            for u in range(8):
                fire(l0 + u, u, sems[u])
            return carry

        lax.fori_loop(0, _MAXLEN // 8, fire_body, 0)

        def drain_body(ll, carry):
            for u in range(8):
                wait(u, sems[u])
            return carry

        lax.fori_loop(0, _MAXLEN // 8, drain_body, 0)

        bias = tuple(b_v[pl.ds(c * 16, 16)] for c in range(4))

        def finish_body(i, carry):
            for c in range(4):
                s01 = (acc_v[0, i, pl.ds(c * 16, 16)]
                       + acc_v[1, i, pl.ds(c * 16, 16)])
                s23 = (acc_v[2, i, pl.ds(c * 16, 16)]
                       + acc_v[3, i, pl.ds(c * 16, 16)])
                s45 = (acc_v[4, i, pl.ds(c * 16, 16)]
                       + acc_v[5, i, pl.ds(c * 16, 16)])
                s67 = (acc_v[6, i, pl.ds(c * 16, 16)]
                       + acc_v[7, i, pl.ds(c * 16, 16)])
                s = (s01 + s23) + (s45 + s67)
                out_v[i, pl.ds(c * 16, 16)] = s * inv_len + bias[c]
            return carry

        lax.fori_loop(0, _BPW, finish_body, 0)
        pltpu.sync_copy(out_v, out_hbm.at[pl.ds(base, _BPW)])

    return mean_kernel


_MEAN_KERNEL = _make_mean_kernel()


def kernel(x, table, fc_w, fc_b):
    wt = fc_w.T
    z = jnp.zeros((_EMBED, _EMBED), jnp.float32)
    w_blk = jnp.block([[wt, z], [z, wt]])
    p_packed = _project(table.T, w_blk)
    p_rows = p_packed.reshape(2 * _PROWS, _LABELS)
    b = x // _W
    s = x - b * _W
    x2 = (((b >> 1) * _W + s) << 1) + (b & 1)
    return _MEAN_KERNEL(x2.T, p_rows, fc_b)


# speedup vs baseline: 1.0767x; 1.0240x over previous
import functools

import jax
import jax.numpy as jnp
from jax import lax
from jax.experimental import pallas as pl
from jax.experimental.pallas import tpu as pltpu
from jax.experimental.pallas import tpu_sc as plsc

_VOCAB = 1000000
_EMBED = 64
_MAXLEN = 200
_LABELS = 64
_BATCH = 4096

_NC, _NS = 2, 16
_NW = _NC * _NS
_BPW = _BATCH // _NW
_C0, _C1 = 104, 96

_W = 15360
_NBLK = -(-_VOCAB // _W)
_NPAIR = -(-_NBLK // 2)
_PROWS = _NPAIR * _W


def _proj_body(tlo_ref, thi_ref, w_ref, o_ref):
    t2 = jnp.concatenate([tlo_ref[...], thi_ref[...]], axis=0)
    o_ref[...] = lax.dot_general(
        t2, w_ref[...], (((0,), (0,)), ((), ())),
        preferred_element_type=jnp.float32)


def _project(table_t, w_blk):
    return pl.pallas_call(
        _proj_body,
        grid=(_NPAIR,),
        in_specs=[
            pl.BlockSpec((_EMBED, _W), lambda i: (0, 2 * i)),
            pl.BlockSpec(
                (_EMBED, _W),
                lambda i: (0, jnp.minimum(2 * i + 1, _NBLK - 1))),
            pl.BlockSpec((2 * _EMBED, 2 * _EMBED), lambda i: (0, 0)),
        ],
        out_specs=pl.BlockSpec((_W, 2 * _EMBED), lambda i: (i, 0)),
        out_shape=jax.ShapeDtypeStruct((_PROWS, 2 * _EMBED), jnp.float32),
    )(table_t, table_t, w_blk)


def _make_mean_kernel():
    mesh = plsc.VectorSubcoreMesh(core_axis_name="c", subcore_axis_name="s")

    @functools.partial(
        pl.kernel,
        out_type=jax.ShapeDtypeStruct((_BATCH, _LABELS), jnp.float32),
        mesh=mesh,
        compiler_params=pltpu.CompilerParams(use_tc_tiling_on_sc=False),
        scratch_types=[
            pltpu.VMEM((_MAXLEN, _BPW), jnp.int32),
            pltpu.VMEM((8, _BPW, _LABELS), jnp.float32),
            pltpu.VMEM((_BPW, _LABELS), jnp.float32),
            pltpu.VMEM((_LABELS,), jnp.float32),
            pltpu.SemaphoreType.DMA,
            pltpu.SemaphoreType.DMA,
            pltpu.SemaphoreType.DMA,
            pltpu.SemaphoreType.DMA,
            pltpu.SemaphoreType.DMA,
            pltpu.SemaphoreType.DMA,
            pltpu.SemaphoreType.DMA,
            pltpu.SemaphoreType.DMA,
        ],
    )
    def mean_kernel(x_hbm, p_hbm, b_hbm, out_hbm,
                    idx_v, acc_v, out_v, b_v,
                    sem0, sem1, sem2, sem3, sem4, sem5, sem6, sem7):
        wid = lax.axis_index("s") * _NC + lax.axis_index("c")
        base = wid * _BPW
        pltpu.sync_copy(b_hbm, b_v)
        pltpu.sync_copy(x_hbm.at[:, pl.ds(base, _BPW)], idx_v)

        inv_len = jnp.float32(1.0 / _MAXLEN)
        zero = jnp.zeros((16,), jnp.float32)

        def zero_body(i, carry):
            for q in range(8):
                for c in range(4):
                    acc_v[q, i, pl.ds(c * 16, 16)] = zero
            return carry

        lax.fori_loop(0, _BPW, zero_body, 0)

        def fire(l, buf, sem):
            pltpu.async_copy(
                p_hbm.at[idx_v.at[l]], acc_v.at[buf], sem, add=True)

        def wait(buf, sem):
            pltpu.make_async_copy(
                p_hbm.at[idx_v.at[0]], acc_v.at[buf], sem).wait()

        sems = (sem0, sem1, sem2, sem3, sem4, sem5, sem6, sem7)

        def fire_body(ll, carry):
            l0 = ll * 8
            for u in range(8):
                fire(l0 + u, u, sems[u])
            return carry

        lax.fori_loop(0, _MAXLEN // 8, fire_body, 0)

        def drain_body(ll, carry):
            for u in range(8):
                wait(u, sems[u])
            return carry

        lax.fori_loop(0, _MAXLEN // 8, drain_body, 0)

        bias = tuple(b_v[pl.ds(c * 16, 16)] for c in range(4))

        def finish_body(i, carry):
            for c in range(4):
                s01 = (acc_v[0, i, pl.ds(c * 16, 16)]
                       + acc_v[1, i, pl.ds(c * 16, 16)])
                s23 = (acc_v[2, i, pl.ds(c * 16, 16)]
                       + acc_v[3, i, pl.ds(c * 16, 16)])
                s45 = (acc_v[4, i, pl.ds(c * 16, 16)]
                       + acc_v[5, i, pl.ds(c * 16, 16)])
                s67 = (acc_v[6, i, pl.ds(c * 16, 16)]
                       + acc_v[7, i, pl.ds(c * 16, 16)])
                s = (s01 + s23) + (s45 + s67)
                out_v[i, pl.ds(c * 16, 16)] = s * inv_len + bias[c]
            return carry

        lax.fori_loop(0, _BPW, finish_body, 0)
        pltpu.sync_copy(out_v, out_hbm.at[pl.ds(base, _BPW)])

    return mean_kernel


_MEAN_KERNEL = _make_mean_kernel()


def kernel(x, table, fc_w, fc_b):
    wt = fc_w.T
    z = jnp.zeros((_EMBED, _EMBED), jnp.float32)
    w_blk = jnp.block([[wt, z], [z, wt]])
    p_packed = _project(table.T, w_blk)
    p_rows = p_packed.reshape(2 * _PROWS, _LABELS)
    b = x // _W
    s = x - b * _W
    x2 = (((b >> 1) * _W + s) << 1) + (b & 1)
    return _MEAN_KERNEL(x2.T, p_rows, fc_b)


# speedup vs baseline: 1.0775x; 1.0008x over previous
import functools

import jax
import jax.numpy as jnp
from jax import lax
from jax.experimental import pallas as pl
from jax.experimental.pallas import tpu as pltpu
from jax.experimental.pallas import tpu_sc as plsc

_VOCAB = 1000000
_EMBED = 64
_MAXLEN = 200
_LABELS = 64
_BATCH = 4096

_NC, _NS = 2, 16
_NW = _NC * _NS
_BPW = _BATCH // _NW
_C0, _C1 = 104, 96

_W = 23040
_NBLK = -(-_VOCAB // _W)
_NPAIR = -(-_NBLK // 2)
_PROWS = _NPAIR * _W


def _proj_body(tlo_ref, thi_ref, w_ref, o_ref):
    t2 = jnp.concatenate([tlo_ref[...], thi_ref[...]], axis=0)
    o_ref[...] = lax.dot_general(
        t2, w_ref[...], (((0,), (0,)), ((), ())),
        preferred_element_type=jnp.float32)


def _project(table_t, w_blk):
    return pl.pallas_call(
        _proj_body,
        grid=(_NPAIR,),
        in_specs=[
            pl.BlockSpec((_EMBED, _W), lambda i: (0, 2 * i)),
            pl.BlockSpec(
                (_EMBED, _W),
                lambda i: (0, jnp.minimum(2 * i + 1, _NBLK - 1))),
            pl.BlockSpec((2 * _EMBED, 2 * _EMBED), lambda i: (0, 0)),
        ],
        out_specs=pl.BlockSpec((_W, 2 * _EMBED), lambda i: (i, 0)),
        out_shape=jax.ShapeDtypeStruct((_PROWS, 2 * _EMBED), jnp.float32),
    )(table_t, table_t, w_blk)


def _make_mean_kernel():
    mesh = plsc.VectorSubcoreMesh(core_axis_name="c", subcore_axis_name="s")

    @functools.partial(
        pl.kernel,
        out_type=jax.ShapeDtypeStruct((_BATCH, _LABELS), jnp.float32),
        mesh=mesh,
        compiler_params=pltpu.CompilerParams(use_tc_tiling_on_sc=False),
        scratch_types=[
            pltpu.VMEM((_MAXLEN, _BPW), jnp.int32),
            pltpu.VMEM((8, _BPW, _LABELS), jnp.float32),
            pltpu.VMEM((_BPW, _LABELS), jnp.float32),
            pltpu.VMEM((_LABELS,), jnp.float32),
            pltpu.SemaphoreType.DMA,
            pltpu.SemaphoreType.DMA,
            pltpu.SemaphoreType.DMA,
            pltpu.SemaphoreType.DMA,
            pltpu.SemaphoreType.DMA,
            pltpu.SemaphoreType.DMA,
            pltpu.SemaphoreType.DMA,
            pltpu.SemaphoreType.DMA,
        ],
    )
    def mean_kernel(x_hbm, p_hbm, b_hbm, out_hbm,
                    idx_v, acc_v, out_v, b_v,
                    sem0, sem1, sem2, sem3, sem4, sem5, sem6, sem7):
        wid = lax.axis_index("s") * _NC + lax.axis_index("c")
        base = wid * _BPW
        pltpu.sync_copy(b_hbm, b_v)
        pltpu.sync_copy(x_hbm.at[:, pl.ds(base, _BPW)], idx_v)

        inv_len = jnp.float32(1.0 / _MAXLEN)
        zero = jnp.zeros((16,), jnp.float32)

        def zero_body(i, carry):
            for q in range(8):
                for c in range(4):
                    acc_v[q, i, pl.ds(c * 16, 16)] = zero
            return carry

        lax.fori_loop(0, _BPW, zero_body, 0)

        def fire(l, buf, sem):
            pltpu.async_copy(
                p_hbm.at[idx_v.at[l]], acc_v.at[buf], sem, add=True)

        def wait(buf, sem):
            pltpu.make_async_copy(
                p_hbm.at[idx_v.at[0]], acc_v.at[buf], sem).wait()

        sems = (sem0, sem1, sem2, sem3, sem4, sem5, sem6, sem7)

        def fire_body(ll, carry):
            l0 = ll * 8
            for u in range(8):
                fire(l0 + u, u, sems[u])
            return carry

        lax.fori_loop(0, _MAXLEN // 8, fire_body, 0)

        def drain_body(ll, carry):
            for u in range(8):
                wait(u, sems[u])
            return carry

        lax.fori_loop(0, _MAXLEN // 8, drain_body, 0)

        bias = tuple(b_v[pl.ds(c * 16, 16)] for c in range(4))

        def finish_body(i, carry):
            for c in range(4):
                s01 = (acc_v[0, i, pl.ds(c * 16, 16)]
                       + acc_v[1, i, pl.ds(c * 16, 16)])
                s23 = (acc_v[2, i, pl.ds(c * 16, 16)]
                       + acc_v[3, i, pl.ds(c * 16, 16)])
                s45 = (acc_v[4, i, pl.ds(c * 16, 16)]
                       + acc_v[5, i, pl.ds(c * 16, 16)])
                s67 = (acc_v[6, i, pl.ds(c * 16, 16)]
                       + acc_v[7, i, pl.ds(c * 16, 16)])
                s = (s01 + s23) + (s45 + s67)
                out_v[i, pl.ds(c * 16, 16)] = s * inv_len + bias[c]
            return carry

        lax.fori_loop(0, _BPW, finish_body, 0)
        pltpu.sync_copy(out_v, out_hbm.at[pl.ds(base, _BPW)])

    return mean_kernel


_MEAN_KERNEL = _make_mean_kernel()


def kernel(x, table, fc_w, fc_b):
    wt = fc_w.T
    z = jnp.zeros((_EMBED, _EMBED), jnp.float32)
    w_blk = jnp.block([[wt, z], [z, wt]])
    p_packed = _project(table.T, w_blk)
    p_rows = p_packed.reshape(2 * _PROWS, _LABELS)
    b = x // _W
    s = x - b * _W
    x2 = (((b >> 1) * _W + s) << 1) + (b & 1)
    return _MEAN_KERNEL(x2.T, p_rows, fc_b)
